# Initial kernel scaffold; baseline (speedup 1.0000x reference)
import jax
import jax.numpy as jnp
from jax.experimental import pallas as pl


def kernel(x, edge_index, w, W1, b1, W2, b2):
    raise NotImplementedError("write your pallas kernel here")


# speedup vs baseline: 3.2838x; 3.2838x over previous
import dataclasses
import functools

import jax
import jax.numpy as jnp
from jax import lax
from jax.experimental import pallas as pl
from jax.experimental.pallas import tpu as pltpu
from jax.experimental.pallas import tpu_sc as plsc

_MIN_NORM = 1e-15
_EPS = 4e-3


def _artanh(x):
    x = jnp.clip(x, -1.0 + 1e-7, 1.0 - 1e-7)
    return 0.5 * jnp.log((1.0 + x) / (1.0 - x))


def _norm(x):
    return jnp.clip(
        jnp.sqrt(jnp.sum(x * x, axis=-1, keepdims=True)), _MIN_NORM, None
    )


def _proj(x):
    norm = _norm(x)
    maxnorm = 1.0 - _EPS
    return jnp.where(norm > maxnorm, x / norm * maxnorm, x)


def _expmap0(u):
    u_norm = _norm(u)
    return jnp.tanh(u_norm) * u / u_norm


def _logmap0(p):
    p_norm = _norm(p)
    return _artanh(p_norm) * p / p_norm


def _mobius_add(x, y):
    x2 = jnp.sum(x * x, axis=-1, keepdims=True)
    y2 = jnp.sum(y * y, axis=-1, keepdims=True)
    xy = jnp.sum(x * y, axis=-1, keepdims=True)
    num = (1.0 + 2.0 * xy + y2) * x + (1.0 - x2) * y
    denom = 1.0 + 2.0 * xy + x2 * y2
    return num / jnp.clip(denom, _MIN_NORM, None)


def _hyp_linear(x, W, b):
    x_norm = _norm(x)
    mx = lax.dot_general(
        x, W, (((1,), (1,)), ((), ())), preferred_element_type=jnp.float32
    )
    mx_norm = _norm(mx)
    res = jnp.tanh(mx_norm / x_norm * _artanh(x_norm)) * mx / mx_norm
    allzero = jnp.sum(jnp.abs(mx), axis=-1, keepdims=True) == 0.0
    res = jnp.where(allzero, jnp.zeros_like(res), res)
    res = _proj(res)
    hb = _proj(_expmap0(b))
    return _proj(_mobius_add(res, hb))


def _agg_post_act(p0, p1):
    h = _proj(_expmap0(p0 + p1))
    xt = jax.nn.relu(_logmap0(h))
    return _proj(_expmap0(xt))


def _tc_pre_body(x_ref, w1_ref, b1_ref, xt1_ref):
    x = x_ref[...]
    h0 = _proj(_expmap0(x))
    res = _hyp_linear(h0, w1_ref[...], b1_ref[...])
    xt1_ref[...] = _logmap0(res)


def _tc_mid_body(p0_ref, p1_ref, w2_ref, b2_ref, h1_ref, xt2_ref):
    h1 = _agg_post_act(p0_ref[...], p1_ref[...])
    h1_ref[...] = h1
    res = _hyp_linear(h1, w2_ref[...], b2_ref[...])
    xt2_ref[...] = _logmap0(res)


def _tc_post_body(p0_ref, p1_ref, h2_ref):
    h2_ref[...] = _agg_post_act(p0_ref[...], p1_ref[...])


def _row_blocked(body, n_out, N, D, BR):
    grid = (N // BR,)
    row_spec = pl.BlockSpec((BR, D), lambda i: (i, 0))
    full_spec = pl.BlockSpec((D, D), lambda i: (0, 0))
    bias_spec = pl.BlockSpec((1, D), lambda i: (0, 0))
    specs = {
        "row": row_spec,
        "mat": full_spec,
        "bias": bias_spec,
    }
    return grid, specs


def _tc_pre(x, W1, b1, BR=400):
    N, D = x.shape
    grid, sp = _row_blocked(_tc_pre_body, 1, N, D, BR)
    return pl.pallas_call(
        _tc_pre_body,
        grid=grid,
        in_specs=[sp["row"], sp["mat"], sp["bias"]],
        out_specs=sp["row"],
        out_shape=jax.ShapeDtypeStruct((N, D), jnp.float32),
    )(x, W1, b1.reshape(1, D))


def _tc_mid(p0, p1, W2, b2, BR=400):
    N, D = p0.shape
    grid, sp = _row_blocked(_tc_mid_body, 2, N, D, BR)
    return pl.pallas_call(
        _tc_mid_body,
        grid=grid,
        in_specs=[sp["row"], sp["row"], sp["mat"], sp["bias"]],
        out_specs=[sp["row"], sp["row"]],
        out_shape=[
            jax.ShapeDtypeStruct((N, D), jnp.float32),
            jax.ShapeDtypeStruct((N, D), jnp.float32),
        ],
    )(p0, p1, W2, b2.reshape(1, D))


def _tc_post(p0, p1, BR=400):
    N, D = p0.shape
    grid, sp = _row_blocked(_tc_post_body, 1, N, D, BR)
    return pl.pallas_call(
        _tc_post_body,
        grid=grid,
        in_specs=[sp["row"], sp["row"]],
        out_specs=sp["row"],
        out_shape=jax.ShapeDtypeStruct((N, D), jnp.float32),
    )(p0, p1)


_NC = 2
_NS = 16
_L = 16


@functools.lru_cache(maxsize=None)
def _make_sc_agg(N, D, E):
    NW = _NC * _NS
    e_per_tile = E // NW
    K = 80
    n_chunks = e_per_tile // K
    ZR = 16
    RPT = (N // _NS) // ZR * ZR
    NB = RPT // ZR
    last_rows = N - 15 * RPT
    NB_LAST = last_rows // ZR
    assert e_per_tile * NW == E and n_chunks * K == e_per_tile
    assert NB_LAST * ZR == last_rows and NB_LAST >= NB

    mesh = plsc.VectorSubcoreMesh(core_axis_name="c", subcore_axis_name="s")
    cp = pltpu.CompilerParams()
    if "needs_layout_passes" in pltpu.CompilerParams.__dataclass_fields__:
        cp = dataclasses.replace(cp, needs_layout_passes=False)

    @functools.partial(
        pl.kernel,
        out_type=jax.ShapeDtypeStruct((_NC, N, D), jnp.float32),
        mesh=mesh,
        compiler_params=cp,
        scratch_types=[
            pltpu.VMEM((K,), jnp.int32),
            pltpu.VMEM((K,), jnp.int32),
            pltpu.VMEM((K,), jnp.float32),
            pltpu.VMEM((K, D), jnp.float32),
            pltpu.VMEM((ZR, D), jnp.float32),
            pltpu.VMEM_SHARED((N, D), jnp.float32),
            pltpu.SemaphoreType.DMA,
        ],
    )
    def agg(table_hbm, src_hbm, dst_hbm, w_hbm, out_hbm,
            src_v, dst_v, w_v, rows_v, zero_v, acc_sh, sem):
        c = lax.axis_index("c")
        s = lax.axis_index("s")
        wid = c * _NS + s
        zvec = jnp.zeros((_L,), jnp.float32)

        @pl.loop(0, ZR)
        def _zrow(r):
            for j in range(D // _L):
                zero_v[r, pl.ds(j * _L, _L)] = zvec

        @pl.loop(0, NB)
        def _zcp(k):
            ro = s * RPT + k * ZR
            pltpu.sync_copy(zero_v, acc_sh.at[pl.ds(ro, ZR)])

        @pl.when(s == _NS - 1)
        def _ztail():
            @pl.loop(NB, NB_LAST)
            def _zcp2(k):
                ro = (_NS - 1) * RPT + k * ZR
                pltpu.sync_copy(zero_v, acc_sh.at[pl.ds(ro, ZR)])

        plsc.subcore_barrier()

        @pl.loop(0, n_chunks)
        def _chunk(i):
            base = wid * e_per_tile + i * K
            pltpu.sync_copy(src_hbm.at[pl.ds(base, K)], src_v)
            pltpu.sync_copy(dst_hbm.at[pl.ds(base, K)], dst_v)
            pltpu.sync_copy(w_hbm.at[pl.ds(base, K)], w_v)
            pltpu.async_copy(table_hbm.at[src_v], rows_v, sem).wait()

            @pl.loop(0, K)
            def _scale(r):
                wsplat = plsc.load_gather(
                    w_v, [jnp.full((_L,), r, jnp.int32)]
                )
                for j in range(D // _L):
                    sl = (r, pl.ds(j * _L, _L))
                    rows_v[sl] = rows_v[sl] * wsplat

            pltpu.async_copy(rows_v, acc_sh.at[dst_v], sem, add=True).wait()

        plsc.subcore_barrier()

        @pl.loop(0, NB)
        def _out(k):
            ro = s * RPT + k * ZR
            pltpu.sync_copy(
                acc_sh.at[pl.ds(ro, ZR)], out_hbm.at[c, pl.ds(ro, ZR)]
            )

        @pl.when(s == _NS - 1)
        def _otail():
            @pl.loop(NB, NB_LAST)
            def _out2(k):
                ro = (_NS - 1) * RPT + k * ZR
                pltpu.sync_copy(
                    acc_sh.at[pl.ds(ro, ZR)], out_hbm.at[c, pl.ds(ro, ZR)]
                )

    return agg


def _sc_agg(table, src, dst, w):
    N, D = table.shape
    E = w.shape[0]
    parts = _make_sc_agg(N, D, E)(table, src, dst, w)
    return parts[0], parts[1]


def kernel(x, edge_index, w, W1, b1, W2, b2):
    src = edge_index[0].astype(jnp.int32)
    dst = edge_index[1].astype(jnp.int32)
    w = w.astype(jnp.float32)

    xt1 = _tc_pre(x, W1, b1)
    p0, p1 = _sc_agg(xt1, src, dst, w)
    h1, xt2 = _tc_mid(p0, p1, W2, b2)
    q0, q1 = _sc_agg(xt2, src, dst, w)
    h2 = _tc_post(q0, q1)
    return jnp.concatenate([x, h1, h2], axis=1)


# speedup vs baseline: 5.7241x; 1.7431x over previous
import dataclasses
import functools

import jax
import jax.numpy as jnp
from jax import lax
from jax.experimental import pallas as pl
from jax.experimental.pallas import tpu as pltpu
from jax.experimental.pallas import tpu_sc as plsc

_MIN_NORM = 1e-15
_EPS = 4e-3


def _artanh(x):
    x = jnp.clip(x, -1.0 + 1e-7, 1.0 - 1e-7)
    return 0.5 * jnp.log((1.0 + x) / (1.0 - x))


def _norm(x):
    return jnp.clip(
        jnp.sqrt(jnp.sum(x * x, axis=-1, keepdims=True)), _MIN_NORM, None
    )


def _proj(x):
    norm = _norm(x)
    maxnorm = 1.0 - _EPS
    return jnp.where(norm > maxnorm, x / norm * maxnorm, x)


def _expmap0(u):
    u_norm = _norm(u)
    return jnp.tanh(u_norm) * u / u_norm


def _logmap0(p):
    p_norm = _norm(p)
    return _artanh(p_norm) * p / p_norm


def _mobius_add(x, y):
    x2 = jnp.sum(x * x, axis=-1, keepdims=True)
    y2 = jnp.sum(y * y, axis=-1, keepdims=True)
    xy = jnp.sum(x * y, axis=-1, keepdims=True)
    num = (1.0 + 2.0 * xy + y2) * x + (1.0 - x2) * y
    denom = 1.0 + 2.0 * xy + x2 * y2
    return num / jnp.clip(denom, _MIN_NORM, None)


def _hyp_linear(x, W, b):
    x_norm = _norm(x)
    mx = lax.dot_general(
        x, W, (((1,), (1,)), ((), ())), preferred_element_type=jnp.float32
    )
    mx_norm = _norm(mx)
    res = jnp.tanh(mx_norm / x_norm * _artanh(x_norm)) * mx / mx_norm
    allzero = jnp.sum(jnp.abs(mx), axis=-1, keepdims=True) == 0.0
    res = jnp.where(allzero, jnp.zeros_like(res), res)
    res = _proj(res)
    hb = _proj(_expmap0(b))
    return _proj(_mobius_add(res, hb))


def _agg_post_act(p0, p1):
    h = _proj(_expmap0(p0 + p1))
    xt = jax.nn.relu(_logmap0(h))
    return _proj(_expmap0(xt))


def _tc_pre_body(x_ref, w1_ref, b1_ref, xt1_ref):
    x = x_ref[...]
    h0 = _proj(_expmap0(x))
    res = _hyp_linear(h0, w1_ref[...], b1_ref[...])
    xt1_ref[...] = _logmap0(res)


def _tc_mid_body(p0_ref, p1_ref, w2_ref, b2_ref, h1_ref, xt2_ref):
    h1 = _agg_post_act(p0_ref[...], p1_ref[...])
    h1_ref[...] = h1
    res = _hyp_linear(h1, w2_ref[...], b2_ref[...])
    xt2_ref[...] = _logmap0(res)


def _tc_post_body(p0_ref, p1_ref, h2_ref):
    h2_ref[...] = _agg_post_act(p0_ref[...], p1_ref[...])


def _row_blocked(body, n_out, N, D, BR):
    grid = (N // BR,)
    row_spec = pl.BlockSpec((BR, D), lambda i: (i, 0))
    full_spec = pl.BlockSpec((D, D), lambda i: (0, 0))
    bias_spec = pl.BlockSpec((1, D), lambda i: (0, 0))
    specs = {
        "row": row_spec,
        "mat": full_spec,
        "bias": bias_spec,
    }
    return grid, specs


def _tc_pre(x, W1, b1, BR=400):
    N, D = x.shape
    grid, sp = _row_blocked(_tc_pre_body, 1, N, D, BR)
    return pl.pallas_call(
        _tc_pre_body,
        grid=grid,
        in_specs=[sp["row"], sp["mat"], sp["bias"]],
        out_specs=sp["row"],
        out_shape=jax.ShapeDtypeStruct((N, D), jnp.float32),
    )(x, W1, b1.reshape(1, D))


def _tc_mid(p0, p1, W2, b2, BR=400):
    N, D = p0.shape
    grid, sp = _row_blocked(_tc_mid_body, 2, N, D, BR)
    return pl.pallas_call(
        _tc_mid_body,
        grid=grid,
        in_specs=[sp["row"], sp["row"], sp["mat"], sp["bias"]],
        out_specs=[sp["row"], sp["row"]],
        out_shape=[
            jax.ShapeDtypeStruct((N, D), jnp.float32),
            jax.ShapeDtypeStruct((N, D), jnp.float32),
        ],
    )(p0, p1, W2, b2.reshape(1, D))


def _tc_post(p0, p1, BR=400):
    N, D = p0.shape
    grid, sp = _row_blocked(_tc_post_body, 1, N, D, BR)
    return pl.pallas_call(
        _tc_post_body,
        grid=grid,
        in_specs=[sp["row"], sp["row"]],
        out_specs=sp["row"],
        out_shape=jax.ShapeDtypeStruct((N, D), jnp.float32),
    )(p0, p1)


_NC = 2
_NS = 16
_L = 16


@functools.lru_cache(maxsize=None)
def _make_sc_agg(N, D, E):
    NW = _NC * _NS
    e_per_tile = E // NW
    K = 80
    n_chunks = e_per_tile // K
    ZR = 16
    RPT = (N // _NS) // ZR * ZR
    NB = RPT // ZR
    last_rows = N - 15 * RPT
    NB_LAST = last_rows // ZR
    assert e_per_tile * NW == E and n_chunks * K == e_per_tile
    assert NB_LAST * ZR == last_rows and NB_LAST >= NB

    mesh = plsc.VectorSubcoreMesh(core_axis_name="c", subcore_axis_name="s")
    cp = pltpu.CompilerParams()
    if "needs_layout_passes" in pltpu.CompilerParams.__dataclass_fields__:
        cp = dataclasses.replace(cp, needs_layout_passes=False)

    @functools.partial(
        pl.kernel,
        out_type=jax.ShapeDtypeStruct((_NC, N, D), jnp.float32),
        mesh=mesh,
        compiler_params=cp,
        scratch_types=[
            pltpu.VMEM((2, K), jnp.int32),
            pltpu.VMEM((2, K), jnp.int32),
            pltpu.VMEM((e_per_tile,), jnp.float32),
            pltpu.VMEM((K, D), jnp.float32),
            pltpu.VMEM((K, D), jnp.float32),
            pltpu.VMEM((ZR, D), jnp.float32),
            pltpu.VMEM_SHARED((N, D), jnp.float32),
            pltpu.SemaphoreType.DMA,
            pltpu.SemaphoreType.DMA,
            pltpu.SemaphoreType.DMA,
            pltpu.SemaphoreType.DMA,
        ],
    )
    def agg(table_hbm, sd_hbm, w_hbm, out_hbm,
            sd0, sd1, w_all, rows0, rows1, zero_v, acc_sh,
            gsem0, gsem1, ssem0, ssem1):
        c = lax.axis_index("c")
        s = lax.axis_index("s")
        wid = c * _NS + s
        zvec = jnp.zeros((_L,), jnp.float32)
        rows = (rows0, rows1)
        sd = (sd0, sd1)
        gsems = (gsem0, gsem1)
        ssems = (ssem0, ssem1)

        @pl.loop(0, ZR)
        def _zrow(r):
            for j in range(D // _L):
                zero_v[r, pl.ds(j * _L, _L)] = zvec

        @pl.loop(0, NB)
        def _zcp(k):
            ro = s * RPT + k * ZR
            pltpu.sync_copy(zero_v, acc_sh.at[pl.ds(ro, ZR)])

        @pl.when(s == _NS - 1)
        def _ztail():
            @pl.loop(NB, NB_LAST)
            def _zcp2(k):
                ro = (_NS - 1) * RPT + k * ZR
                pltpu.sync_copy(zero_v, acc_sh.at[pl.ds(ro, ZR)])

        pltpu.sync_copy(w_hbm.at[pl.ds(wid * e_per_tile, e_per_tile)], w_all)

        plsc.subcore_barrier()

        def start_gather(i, b):
            pltpu.sync_copy(sd_hbm.at[wid, i], sd[b])
            pltpu.async_copy(table_hbm.at[sd[b].at[0]], rows[b], gsems[b])

        def wait_gather(i, b):
            del i
            pltpu.make_async_copy(
                table_hbm.at[sd[b].at[0]], rows[b], gsems[b]
            ).wait()

        def start_scatter(i, b):
            del i
            pltpu.async_copy(
                rows[b], acc_sh.at[sd[b].at[1]], ssems[b], add=True
            )

        def wait_scatter(i, b):
            del i
            pltpu.make_async_copy(
                rows[b], acc_sh.at[sd[b].at[1]], ssems[b]
            ).wait()

        def scale(i, b):
            buf = rows[b]

            @pl.loop(0, K)
            def _scale(r):
                wsplat = plsc.load_gather(
                    w_all, [jnp.full((_L,), i * K + r, jnp.int32)]
                )
                for j in range(D // _L):
                    sl = (r, pl.ds(j * _L, _L))
                    buf[sl] = buf[sl] * wsplat

        start_gather(0, 0)
        start_gather(1, 1)

        @pl.loop(0, n_chunks - 1, step=2)
        def _chunk(i):
            for b in range(2):
                j = i + b
                wait_gather(j, b)
                scale(j, b)
                start_scatter(j, b)
            for b in range(2):
                j = i + b
                nxt = j + 2

                @pl.when(nxt < n_chunks)
                def _next():
                    wait_scatter(j, b)
                    start_gather(nxt, b)

        if n_chunks % 2 == 1:
            last = n_chunks - 1
            wait_gather(last, 0)
            scale(last, 0)
            start_scatter(last, 0)
            wait_scatter(last, 0)
            wait_scatter(n_chunks - 2, 1)
        else:
            wait_scatter(n_chunks - 2, 0)
            wait_scatter(n_chunks - 1, 1)

        plsc.subcore_barrier()

        @pl.loop(0, NB)
        def _out(k):
            ro = s * RPT + k * ZR
            pltpu.sync_copy(
                acc_sh.at[pl.ds(ro, ZR)], out_hbm.at[c, pl.ds(ro, ZR)]
            )

        @pl.when(s == _NS - 1)
        def _otail():
            @pl.loop(NB, NB_LAST)
            def _out2(k):
                ro = (_NS - 1) * RPT + k * ZR
                pltpu.sync_copy(
                    acc_sh.at[pl.ds(ro, ZR)], out_hbm.at[c, pl.ds(ro, ZR)]
                )

    return agg


def _sc_agg(table, src, dst, w):
    N, D = table.shape
    E = w.shape[0]
    NW = _NC * _NS
    K = 80
    n_chunks = E // NW // K
    sd = jnp.stack(
        [src.reshape(NW, n_chunks, K), dst.reshape(NW, n_chunks, K)], axis=2
    )
    parts = _make_sc_agg(N, D, E)(table, sd, w)
    return parts[0], parts[1]


def kernel(x, edge_index, w, W1, b1, W2, b2):
    src = edge_index[0].astype(jnp.int32)
    dst = edge_index[1].astype(jnp.int32)
    w = w.astype(jnp.float32)

    xt1 = _tc_pre(x, W1, b1)
    p0, p1 = _sc_agg(xt1, src, dst, w)
    h1, xt2 = _tc_mid(p0, p1, W2, b2)
    q0, q1 = _sc_agg(xt2, src, dst, w)
    h2 = _tc_post(q0, q1)
    return jnp.concatenate([x, h1, h2], axis=1)


# speedup vs baseline: 6.4681x; 1.1300x over previous
import dataclasses
import functools

import jax
import jax.numpy as jnp
from jax import lax
from jax.experimental import pallas as pl
from jax.experimental.pallas import tpu as pltpu
from jax.experimental.pallas import tpu_sc as plsc

_MIN_NORM = 1e-15
_EPS = 4e-3


def _artanh(x):
    x = jnp.clip(x, -1.0 + 1e-7, 1.0 - 1e-7)
    return 0.5 * jnp.log((1.0 + x) / (1.0 - x))


def _norm(x):
    return jnp.clip(
        jnp.sqrt(jnp.sum(x * x, axis=-1, keepdims=True)), _MIN_NORM, None
    )


def _proj(x):
    norm = _norm(x)
    maxnorm = 1.0 - _EPS
    return jnp.where(norm > maxnorm, x / norm * maxnorm, x)


def _expmap0(u):
    u_norm = _norm(u)
    return jnp.tanh(u_norm) * u / u_norm


def _logmap0(p):
    p_norm = _norm(p)
    return _artanh(p_norm) * p / p_norm


def _mobius_add(x, y):
    x2 = jnp.sum(x * x, axis=-1, keepdims=True)
    y2 = jnp.sum(y * y, axis=-1, keepdims=True)
    xy = jnp.sum(x * y, axis=-1, keepdims=True)
    num = (1.0 + 2.0 * xy + y2) * x + (1.0 - x2) * y
    denom = 1.0 + 2.0 * xy + x2 * y2
    return num / jnp.clip(denom, _MIN_NORM, None)


def _hyp_linear(x, W, b):
    x_norm = _norm(x)
    mx = lax.dot_general(
        x, W, (((1,), (1,)), ((), ())), preferred_element_type=jnp.float32
    )
    mx_norm = _norm(mx)
    res = jnp.tanh(mx_norm / x_norm * _artanh(x_norm)) * mx / mx_norm
    allzero = jnp.sum(jnp.abs(mx), axis=-1, keepdims=True) == 0.0
    res = jnp.where(allzero, jnp.zeros_like(res), res)
    res = _proj(res)
    hb = _proj(_expmap0(b))
    return _proj(_mobius_add(res, hb))


def _agg_post_act(p0, p1):
    h = _proj(_expmap0(p0 + p1))
    xt = jax.nn.relu(_logmap0(h))
    return _proj(_expmap0(xt))


def _tc_pre_body(x_ref, w1_ref, b1_ref, xt1_ref):
    x = x_ref[...]
    h0 = _proj(_expmap0(x))
    res = _hyp_linear(h0, w1_ref[...], b1_ref[...])
    xt1_ref[...] = _logmap0(res)


def _tc_mid_body(p0_ref, p1_ref, w2_ref, b2_ref, h1_ref, xt2_ref):
    h1 = _agg_post_act(p0_ref[...], p1_ref[...])
    h1_ref[...] = h1
    res = _hyp_linear(h1, w2_ref[...], b2_ref[...])
    xt2_ref[...] = _logmap0(res)


def _tc_post_body(p0_ref, p1_ref, h2_ref):
    h2_ref[...] = _agg_post_act(p0_ref[...], p1_ref[...])


def _row_blocked(body, n_out, N, D, BR):
    grid = (N // BR,)
    row_spec = pl.BlockSpec((BR, D), lambda i: (i, 0))
    full_spec = pl.BlockSpec((D, D), lambda i: (0, 0))
    bias_spec = pl.BlockSpec((1, D), lambda i: (0, 0))
    specs = {
        "row": row_spec,
        "mat": full_spec,
        "bias": bias_spec,
    }
    return grid, specs


def _tc_pre(x, W1, b1, BR=400):
    N, D = x.shape
    grid, sp = _row_blocked(_tc_pre_body, 1, N, D, BR)
    return pl.pallas_call(
        _tc_pre_body,
        grid=grid,
        in_specs=[sp["row"], sp["mat"], sp["bias"]],
        out_specs=sp["row"],
        out_shape=jax.ShapeDtypeStruct((N, D), jnp.float32),
    )(x, W1, b1.reshape(1, D))


def _tc_mid(p0, p1, W2, b2, BR=400):
    N, D = p0.shape
    grid, sp = _row_blocked(_tc_mid_body, 2, N, D, BR)
    return pl.pallas_call(
        _tc_mid_body,
        grid=grid,
        in_specs=[sp["row"], sp["row"], sp["mat"], sp["bias"]],
        out_specs=[sp["row"], sp["row"]],
        out_shape=[
            jax.ShapeDtypeStruct((N, D), jnp.float32),
            jax.ShapeDtypeStruct((N, D), jnp.float32),
        ],
    )(p0, p1, W2, b2.reshape(1, D))


def _tc_post(p0, p1, BR=400):
    N, D = p0.shape
    grid, sp = _row_blocked(_tc_post_body, 1, N, D, BR)
    return pl.pallas_call(
        _tc_post_body,
        grid=grid,
        in_specs=[sp["row"], sp["row"]],
        out_specs=sp["row"],
        out_shape=jax.ShapeDtypeStruct((N, D), jnp.float32),
    )(p0, p1)


_NC = 2
_NS = 16
_L = 16


@functools.lru_cache(maxsize=None)
def _make_sc_agg(N, D, E):
    NW = _NC * _NS
    e_per_tile = E // NW
    K = 80
    n_chunks = e_per_tile // K
    ZR = 16
    RPT = (N // _NS) // ZR * ZR
    NB = RPT // ZR
    last_rows = N - 15 * RPT
    NB_LAST = last_rows // ZR
    assert e_per_tile * NW == E and n_chunks * K == e_per_tile
    assert NB_LAST * ZR == last_rows and NB_LAST >= NB

    mesh = plsc.VectorSubcoreMesh(core_axis_name="c", subcore_axis_name="s")
    cp = pltpu.CompilerParams()
    if "needs_layout_passes" in pltpu.CompilerParams.__dataclass_fields__:
        cp = dataclasses.replace(cp, needs_layout_passes=False)

    @functools.partial(
        pl.kernel,
        out_type=jax.ShapeDtypeStruct((_NC, N, D), jnp.float32),
        mesh=mesh,
        compiler_params=cp,
        scratch_types=[
            pltpu.VMEM((2, K), jnp.int32),
            pltpu.VMEM((2, K), jnp.int32),
            pltpu.VMEM((e_per_tile,), jnp.float32),
            pltpu.VMEM((K, D), jnp.float32),
            pltpu.VMEM((K, D), jnp.float32),
            pltpu.VMEM((ZR, D), jnp.float32),
            pltpu.VMEM_SHARED((N, D), jnp.float32),
            pltpu.SemaphoreType.DMA,
            pltpu.SemaphoreType.DMA,
            pltpu.SemaphoreType.DMA,
            pltpu.SemaphoreType.DMA,
        ],
    )
    def agg(table_hbm, sd_hbm, w_hbm, out_hbm,
            sd0, sd1, w_all, rows0, rows1, zero_v, acc_sh,
            gsem0, gsem1, ssem0, ssem1):
        c = lax.axis_index("c")
        s = lax.axis_index("s")
        wid = c * _NS + s
        zvec = jnp.zeros((_L,), jnp.float32)
        rows = (rows0, rows1)
        sd = (sd0, sd1)
        gsems = (gsem0, gsem1)
        ssems = (ssem0, ssem1)

        @pl.loop(0, ZR)
        def _zrow(r):
            for j in range(D // _L):
                zero_v[r, pl.ds(j * _L, _L)] = zvec

        @pl.loop(0, NB)
        def _zcp(k):
            ro = s * RPT + k * ZR
            pltpu.sync_copy(zero_v, acc_sh.at[pl.ds(ro, ZR)])

        @pl.when(s == _NS - 1)
        def _ztail():
            @pl.loop(NB, NB_LAST)
            def _zcp2(k):
                ro = (_NS - 1) * RPT + k * ZR
                pltpu.sync_copy(zero_v, acc_sh.at[pl.ds(ro, ZR)])

        pltpu.sync_copy(w_hbm.at[pl.ds(wid * e_per_tile, e_per_tile)], w_all)

        plsc.subcore_barrier()

        def start_gather(i, b):
            pltpu.sync_copy(sd_hbm.at[wid, i], sd[b])
            pltpu.async_copy(table_hbm.at[sd[b].at[0]], rows[b], gsems[b])

        def wait_gather(i, b):
            del i
            pltpu.make_async_copy(
                table_hbm.at[sd[b].at[0]], rows[b], gsems[b]
            ).wait()

        def start_scatter(i, b):
            del i
            pltpu.async_copy(
                rows[b], acc_sh.at[sd[b].at[1]], ssems[b], add=True
            )

        def wait_scatter(i, b):
            del i
            pltpu.make_async_copy(
                rows[b], acc_sh.at[sd[b].at[1]], ssems[b]
            ).wait()

        def scale(i, b):
            buf = rows[b]

            @plsc.parallel_loop(0, K, unroll=8)
            def _scale(r):
                wsplat = plsc.load_gather(
                    w_all, [jnp.full((_L,), i * K + r, jnp.int32)]
                )
                for j in range(D // _L):
                    sl = (r, pl.ds(j * _L, _L))
                    buf[sl] = buf[sl] * wsplat

        start_gather(0, 0)
        start_gather(1, 1)

        @pl.loop(0, n_chunks - 1, step=2)
        def _chunk(i):
            for b in range(2):
                j = i + b
                wait_gather(j, b)
                scale(j, b)
                start_scatter(j, b)
            for b in range(2):
                j = i + b
                nxt = j + 2

                @pl.when(nxt < n_chunks)
                def _next():
                    wait_scatter(j, b)
                    start_gather(nxt, b)

        if n_chunks % 2 == 1:
            last = n_chunks - 1
            wait_gather(last, 0)
            scale(last, 0)
            start_scatter(last, 0)
            wait_scatter(last, 0)
            wait_scatter(n_chunks - 2, 1)
        else:
            wait_scatter(n_chunks - 2, 0)
            wait_scatter(n_chunks - 1, 1)

        plsc.subcore_barrier()

        @pl.loop(0, NB)
        def _out(k):
            ro = s * RPT + k * ZR
            pltpu.sync_copy(
                acc_sh.at[pl.ds(ro, ZR)], out_hbm.at[c, pl.ds(ro, ZR)]
            )

        @pl.when(s == _NS - 1)
        def _otail():
            @pl.loop(NB, NB_LAST)
            def _out2(k):
                ro = (_NS - 1) * RPT + k * ZR
                pltpu.sync_copy(
                    acc_sh.at[pl.ds(ro, ZR)], out_hbm.at[c, pl.ds(ro, ZR)]
                )

    return agg


def _sc_agg(table, src, dst, w):
    N, D = table.shape
    E = w.shape[0]
    NW = _NC * _NS
    K = 80
    n_chunks = E // NW // K
    sd = jnp.stack(
        [src.reshape(NW, n_chunks, K), dst.reshape(NW, n_chunks, K)], axis=2
    )
    parts = _make_sc_agg(N, D, E)(table, sd, w)
    return parts[0], parts[1]


def kernel(x, edge_index, w, W1, b1, W2, b2):
    src = edge_index[0].astype(jnp.int32)
    dst = edge_index[1].astype(jnp.int32)
    w = w.astype(jnp.float32)

    xt1 = _tc_pre(x, W1, b1)
    p0, p1 = _sc_agg(xt1, src, dst, w)
    h1, xt2 = _tc_mid(p0, p1, W2, b2)
    q0, q1 = _sc_agg(xt2, src, dst, w)
    h2 = _tc_post(q0, q1)
    return jnp.concatenate([x, h1, h2], axis=1)
